# baseline (speedup 1.0000x reference)
import functools

import jax
import jax.numpy as jnp
from jax import lax
from jax.experimental import pallas as pl
from jax.experimental.pallas import tpu as pltpu
from jax.experimental.pallas import tpu_sc as plsc

_EMB = 64
_SCALE = 8.0
_NC, _NS, _L = 2, 16, 16
_NW = _NC * _NS
_T = 200
_BATCH = 4096
_BB = _BATCH // _NW

_mesh = plsc.VectorSubcoreMesh(core_axis_name="c", subcore_axis_name="s")


@functools.partial(
    pl.kernel,
    out_type=jax.ShapeDtypeStruct((_T, _EMB, _BATCH), jnp.float32),
    mesh=_mesh,
    scratch_types=[
        pltpu.VMEM((_T, _BB), jnp.int32),
        pltpu.VMEM((2, _BB, _EMB), jnp.float32),
        pltpu.VMEM((_BB // _L, _EMB // _L, _L, 17), jnp.float32),
        pltpu.VMEM((2, _EMB, _BB), jnp.float32),
        pltpu.SemaphoreType.DMA,
        pltpu.SemaphoreType.DMA,
        pltpu.SemaphoreType.DMA,
        pltpu.SemaphoreType.DMA,
    ],
    compiler_params=pltpu.CompilerParams(
        use_tc_tiling_on_sc=False, needs_layout_passes=False
    ),
)
def _emb_lookup(table_hbm, tok_hbm, out_hbm, idx_v, rows_v, scr_v, outt_v,
                gsem0, gsem1, osem0, osem1):
    gsem = (gsem0, gsem1)
    osem = (osem0, osem1)
    wid = lax.axis_index("s") * _NC + lax.axis_index("c")
    b0 = wid * _BB

    pltpu.sync_copy(tok_hbm.at[:, pl.ds(b0, _BB)], idx_v)

    iota = lax.iota(jnp.int32, _L)
    kvecs = [jnp.full((_L,), k, jnp.int32) for k in range(_L)]

    def start_gather(t, b):
        pltpu.make_async_copy(
            table_hbm.at[idx_v.at[t]],
            rows_v.at[b, :, pl.ds(0, _EMB)],
            gsem[b],
        ).start()

    def wait_gather(t, b):
        pltpu.make_async_copy(
            table_hbm.at[idx_v.at[t]],
            rows_v.at[b, :, pl.ds(0, _EMB)],
            gsem[b],
        ).wait()

    def start_out(t, b):
        pltpu.make_async_copy(
            outt_v.at[b], out_hbm.at[t, :, pl.ds(b0, _BB)], osem[b]
        ).start()

    def wait_out(t, b):
        pltpu.make_async_copy(
            outt_v.at[b], out_hbm.at[t, :, pl.ds(b0, _BB)], osem[b]
        ).wait()

    start_gather(0, 0)
    start_gather(1, 1)

    @pl.loop(0, _T, step=2)
    def _pair(t0):
        for b in range(2):
            t = t0 + b
            wait_gather(t, b)

            @pl.when(t >= 2)
            def _():
                wait_out(t - 2, b)

            rows = rows_v.at[b]
            outt = outt_v.at[b]

            for j in range(_BB // _L):
                for cb in range(_EMB // _L):
                    scr = scr_v.at[j, cb]
                    for r in range(_L):
                        scr[r, pl.ds(0, _L)] = rows[
                            j * _L + r, pl.ds(cb * _L, _L)
                        ]
                    for k in range(_L):
                        v = plsc.load_gather(scr, [iota, kvecs[k]])
                        outt[cb * _L + k, pl.ds(j * _L, _L)] = v * _SCALE

            start_out(t, b)

            @pl.when(t + 2 < _T)
            def _():
                start_gather(t + 2, b)

    wait_out(_T - 2, 0)
    wait_out(_T - 1, 1)


def kernel(tokens, table):
    out_t = _emb_lookup(table, tokens.T)
    return jnp.transpose(out_t, (2, 0, 1))


# speedup vs baseline: 1.6842x; 1.6842x over previous
import functools

import jax
import jax.numpy as jnp
from jax import lax
from jax.experimental import pallas as pl
from jax.experimental.pallas import tpu as pltpu
from jax.experimental.pallas import tpu_sc as plsc

_EMB = 64
_SCALE = 8.0
_NC, _NS = 2, 16
_NW = _NC * _NS
_T = 200
_BATCH = 4096
_B = _T * _BATCH
_BPW = _B // _NW
_C = 512
_NCH = _BPW // _C
_HALF = _B // 2

_mesh = plsc.VectorSubcoreMesh(core_axis_name="c", subcore_axis_name="s")


@functools.partial(
    pl.kernel,
    out_type=jax.ShapeDtypeStruct((_HALF, 2 * _EMB), jnp.float32),
    mesh=_mesh,
    scratch_types=[
        pltpu.VMEM((_BPW,), jnp.int32),
        pltpu.VMEM((2, _C, _EMB), jnp.float32),
        pltpu.SemaphoreType.DMA,
        pltpu.SemaphoreType.DMA,
        pltpu.SemaphoreType.DMA,
        pltpu.SemaphoreType.DMA,
    ],
    compiler_params=pltpu.CompilerParams(
        use_tc_tiling_on_sc=False, needs_layout_passes=False
    ),
)
def _sc_gather(table_hbm, idx_hbm, i_hbm, idx_v, rows_v, gs0, gs1, os0, os1):
    gs = (gs0, gs1)
    ws = (os0, os1)
    wid = lax.axis_index("s") * _NC + lax.axis_index("c")
    base = wid * _BPW

    pltpu.sync_copy(idx_hbm.at[pl.ds(base, _BPW)], idx_v)

    def g_copy(k, b):
        return pltpu.make_async_copy(
            table_hbm.at[idx_v.at[pl.ds(k * _C, _C)]], rows_v.at[b], gs[b]
        )

    def w_copy(k, b):
        p0 = base + k * _C
        t0 = p0 // _BATCH
        rem = p0 - t0 * _BATCH
        half = rem // (_BATCH // 2)
        row0 = t0 * (_BATCH // 2) + rem - half * (_BATCH // 2)
        return pltpu.make_async_copy(
            rows_v.at[b],
            i_hbm.at[pl.ds(row0, _C), pl.ds(half * _EMB, _EMB)],
            ws[b],
        )

    g_copy(0, 0).start()
    g_copy(1, 1).start()

    @pl.loop(0, _NCH, step=2)
    def _pair(k0):
        for b in range(2):
            k = k0 + b
            g_copy(k, b).wait()
            w_copy(k, b).start()

            @pl.when(k + 2 < _NCH)
            def _():
                w_copy(k, b).wait()
                g_copy(k + 2, b).start()

    w_copy(_NCH - 2, 0).wait()
    w_copy(_NCH - 1, 1).wait()


@functools.partial(
    pl.pallas_call,
    grid=(_T,),
    in_specs=[pl.BlockSpec((_BATCH // 2, 2 * _EMB), lambda t: (t, 0))],
    out_specs=pl.BlockSpec((1, _EMB, _BATCH), lambda t: (t, 0, 0)),
    out_shape=jax.ShapeDtypeStruct((_T, _EMB, _BATCH), jnp.float32),
)
def _tc_transpose(x_ref, o_ref):
    x = x_ref[...]
    o_ref[0, :, 0 : _BATCH // 2] = x[:, :_EMB].T * _SCALE
    o_ref[0, :, _BATCH // 2 :] = x[:, _EMB:].T * _SCALE


def kernel(tokens, table):
    flat = tokens.T.reshape(-1)
    packed = _sc_gather(table, flat)
    out_t = _tc_transpose(packed)
    return jnp.transpose(out_t, (2, 0, 1))


# speedup vs baseline: 1.7774x; 1.0553x over previous
import functools

import jax
import jax.numpy as jnp
from jax import lax
from jax.experimental import pallas as pl
from jax.experimental.pallas import tpu as pltpu
from jax.experimental.pallas import tpu_sc as plsc

_EMB = 64
_SCALE = 8.0
_NC, _NS = 2, 16
_NW = _NC * _NS
_T = 200
_BATCH = 4096
_BB = _BATCH // _NW
_HB = _BATCH // 2
_NSEG = 4
_TSEG = _T // _NSEG

_mesh = plsc.VectorSubcoreMesh(core_axis_name="c", subcore_axis_name="s")


def _make_sc_gather(t0):
    @functools.partial(
        pl.kernel,
        out_type=jax.ShapeDtypeStruct((_TSEG * _HB, 2 * _EMB), jnp.float32),
        mesh=_mesh,
        scratch_types=[
            pltpu.VMEM((_TSEG, _BB), jnp.int32),
            pltpu.VMEM((2, _BB, _EMB), jnp.float32),
            pltpu.SemaphoreType.DMA,
            pltpu.SemaphoreType.DMA,
            pltpu.SemaphoreType.DMA,
            pltpu.SemaphoreType.DMA,
        ],
        compiler_params=pltpu.CompilerParams(
            use_tc_tiling_on_sc=False, needs_layout_passes=False
        ),
        name=f"sc_gather_seg{t0}",
    )
    def _sc_gather(table_hbm, tok_hbm, i_hbm, idx_v, rows_v, g0, g1, w0, w1):
        gs = (g0, g1)
        ws = (w0, w1)
        wid = lax.axis_index("s") * _NC + lax.axis_index("c")
        b0 = wid * _BB
        half = wid // (_NW // 2)
        boff = b0 - half * _HB
        coloff = half * _EMB

        pltpu.sync_copy(
            tok_hbm.at[pl.ds(t0, _TSEG), pl.ds(b0, _BB)], idx_v
        )

        def g_copy(t, b):
            return pltpu.make_async_copy(
                table_hbm.at[idx_v.at[t]], rows_v.at[b], gs[b]
            )

        def w_copy(t, b):
            return pltpu.make_async_copy(
                rows_v.at[b],
                i_hbm.at[pl.ds(t * _HB + boff, _BB), pl.ds(coloff, _EMB)],
                ws[b],
            )

        g_copy(0, 0).start()
        g_copy(1, 1).start()

        @pl.loop(0, _TSEG, step=2)
        def _pair(tp):
            for b in range(2):
                t = tp + b
                g_copy(t, b).wait()
                w_copy(t, b).start()

                @pl.when(t + 2 < _TSEG)
                def _():
                    w_copy(t, b).wait()
                    g_copy(t + 2, b).start()

        w_copy(_TSEG - 2, 0).wait()
        w_copy(_TSEG - 1, 1).wait()

    return _sc_gather


_SC_GATHERS = [_make_sc_gather(q * _TSEG) for q in range(_NSEG)]


def _tc_body(x_ref, o_ref):
    x = x_ref[...]
    o_ref[0, :, 0:_HB] = x[:, :_EMB].T * _SCALE
    o_ref[0, :, _HB:] = x[:, _EMB:].T * _SCALE


def _tc_body_alias(x_ref, _oprev_ref, o_ref):
    _tc_body(x_ref, o_ref)


_OUT_SHAPE = jax.ShapeDtypeStruct((_T, _EMB, _BATCH), jnp.float32)


def _make_tc_first():
    return pl.pallas_call(
        _tc_body,
        grid=(_TSEG,),
        in_specs=[pl.BlockSpec((_HB, 2 * _EMB), lambda t: (t, 0))],
        out_specs=pl.BlockSpec((1, _EMB, _BATCH), lambda t: (t, 0, 0)),
        out_shape=_OUT_SHAPE,
        name="tc_transpose_seg0",
    )


def _make_tc_alias(q):
    t_base = q * _TSEG
    return pl.pallas_call(
        _tc_body_alias,
        grid=(_TSEG,),
        in_specs=[
            pl.BlockSpec((_HB, 2 * _EMB), lambda t: (t, 0)),
            pl.BlockSpec(memory_space=pl.ANY),
        ],
        out_specs=pl.BlockSpec(
            (1, _EMB, _BATCH), lambda t: (t_base + t, 0, 0)
        ),
        out_shape=_OUT_SHAPE,
        input_output_aliases={1: 0},
        name=f"tc_transpose_seg{q}",
    )


_TC_FIRST = _make_tc_first()
_TC_ALIAS = [_make_tc_alias(q) for q in range(1, _NSEG)]


def kernel(tokens, table):
    tok2d = tokens.T
    segs = [_SC_GATHERS[q](table, tok2d) for q in range(_NSEG)]
    out_t = _TC_FIRST(segs[0])
    for q in range(1, _NSEG):
        out_t = _TC_ALIAS[q - 1](segs[q], out_t)
    return jnp.transpose(out_t, (2, 0, 1))


# speedup vs baseline: 1.9019x; 1.0700x over previous
import functools

import jax
import jax.numpy as jnp
from jax import lax
from jax.experimental import pallas as pl
from jax.experimental.pallas import tpu as pltpu
from jax.experimental.pallas import tpu_sc as plsc

_EMB = 64
_SCALE = 8.0
_NC, _NS = 2, 16
_NW = _NC * _NS
_T = 200
_BATCH = 4096
_BB = _BATCH // _NW
_HB = _BATCH // 2
_NSEG = 4
_TSEG = _T // _NSEG

_mesh = plsc.VectorSubcoreMesh(core_axis_name="c", subcore_axis_name="s")


_VOCAB2 = 2 * 1000000


def _make_sc_gather(t0):
    @functools.partial(
        pl.kernel,
        out_type=jax.ShapeDtypeStruct((_TSEG * _HB, 2 * _EMB), jnp.float32),
        mesh=_mesh,
        scratch_types=[
            pltpu.VMEM((_TSEG, _BB), jnp.int32),
            pltpu.VMEM((2, _BB, _EMB), jnp.float32),
            pltpu.SemaphoreType.DMA,
            pltpu.SemaphoreType.DMA,
            pltpu.SemaphoreType.DMA,
            pltpu.SemaphoreType.DMA,
        ],
        compiler_params=pltpu.CompilerParams(
            use_tc_tiling_on_sc=False, needs_layout_passes=False
        ),
        name=f"sc_gather_seg{t0}",
    )
    def _sc_gather(table_hbm, tok_hbm, i_hbm, idx_v, rows_v, g0, g1, w0, w1):
        gs = (g0, g1)
        ws = (w0, w1)
        wid = lax.axis_index("s") * _NC + lax.axis_index("c")
        b0 = wid * _BB
        half = wid // (_NW // 2)
        boff = b0 - half * _HB
        coloff = half * _EMB

        pltpu.sync_copy(
            tok_hbm.at[pl.ds(t0, _TSEG), pl.ds(b0, _BB)], idx_v
        )

        def g_copy(t, b):
            return pltpu.make_async_copy(
                table_hbm.at[idx_v.at[t]], rows_v.at[b], gs[b]
            )

        def w_copy(t, b):
            return pltpu.make_async_copy(
                rows_v.at[b],
                i_hbm.at[pl.ds(t * _HB + boff, _BB), pl.ds(coloff, _EMB)],
                ws[b],
            )

        g_copy(0, 0).start()
        g_copy(1, 1).start()

        @pl.loop(0, _TSEG, step=2)
        def _pair(tp):
            for b in range(2):
                t = tp + b
                g_copy(t, b).wait()
                w_copy(t, b).start()

                @pl.when(t + 2 < _TSEG)
                def _():
                    w_copy(t, b).wait()
                    g_copy(t + 2, b).start()

        w_copy(_TSEG - 2, 0).wait()
        w_copy(_TSEG - 1, 1).wait()

    return _sc_gather


_SC_GATHERS = [_make_sc_gather(q * _TSEG) for q in range(_NSEG)]


def _tc_body(x_ref, o_ref):
    x = x_ref[...]
    o_ref[0, :, 0:_HB] = x[:, :_EMB].T * _SCALE
    o_ref[0, :, _HB:] = x[:, _EMB:].T * _SCALE


def _tc_body_alias(x_ref, _oprev_ref, o_ref):
    _tc_body(x_ref, o_ref)


_OUT_SHAPE = jax.ShapeDtypeStruct((_T, _EMB, _BATCH), jnp.float32)


def _make_tc_first():
    return pl.pallas_call(
        _tc_body,
        grid=(_TSEG,),
        in_specs=[pl.BlockSpec((_HB, 2 * _EMB), lambda t: (t, 0))],
        out_specs=pl.BlockSpec((1, _EMB, _BATCH), lambda t: (t, 0, 0)),
        out_shape=_OUT_SHAPE,
        name="tc_transpose_seg0",
    )


def _make_tc_alias(q):
    t_base = q * _TSEG
    return pl.pallas_call(
        _tc_body_alias,
        grid=(_TSEG,),
        in_specs=[
            pl.BlockSpec((_HB, 2 * _EMB), lambda t: (t, 0)),
            pl.BlockSpec(memory_space=pl.ANY),
        ],
        out_specs=pl.BlockSpec(
            (1, _EMB, _BATCH), lambda t: (t_base + t, 0, 0)
        ),
        out_shape=_OUT_SHAPE,
        input_output_aliases={1: 0},
        name=f"tc_transpose_seg{q}",
    )


_TC_FIRST = _make_tc_first()
_TC_ALIAS = [_make_tc_alias(q) for q in range(1, _NSEG)]


def kernel(tokens, table):
    tab2 = jnp.pad(table, ((0, 0), (0, _EMB))).reshape(_VOCAB2, _EMB)
    tok2d = tokens.T * 2
    segs = [_SC_GATHERS[q](tab2, tok2d) for q in range(_NSEG)]
    out_t = _TC_FIRST(segs[0])
    for q in range(1, _NSEG):
        out_t = _TC_ALIAS[q - 1](segs[q], out_t)
    return jnp.transpose(out_t, (2, 0, 1))
